# Initial kernel scaffold; baseline (speedup 1.0000x reference)
import jax
import jax.numpy as jnp
from jax.experimental import pallas as pl


def kernel(x, batch, W1, b1, W2):
    raise NotImplementedError("write your pallas kernel here")


# speedup vs baseline: 1.5206x; 1.5206x over previous
import functools

import jax
import jax.numpy as jnp
from jax import lax
from jax.experimental import pallas as pl
from jax.experimental.pallas import tpu as pltpu
from jax.experimental.pallas import tpu_sc as plsc

N_NODES = 100000
NODE_DIM = 128
HIDDEN_DIM = 128
NUM_SEGMENTS = 512

_LOG2 = 0.6931471805599453


_ROWS_PER_TILE = 2000
_N_TILES = N_NODES // _ROWS_PER_TILE


def _mlp_body(x_ref, w1_ref, b1_ref, w2_ref, o_ref):
    h = jnp.dot(x_ref[...], w1_ref[...], preferred_element_type=jnp.float32)
    h = h + b1_ref[...]
    a = jnp.maximum(h, 0.0) + jnp.log1p(jnp.exp(-jnp.abs(h))) - _LOG2
    o_ref[...] = jnp.sum(a * w2_ref[...], axis=1)[None, None, :]


def _mlp_scalars(x, W1, b1, W2):
    b1r = b1.reshape(1, HIDDEN_DIM)
    w2r = W2.reshape(1, HIDDEN_DIM)
    out = pl.pallas_call(
        _mlp_body,
        grid=(_N_TILES,),
        in_specs=[
            pl.BlockSpec((_ROWS_PER_TILE, NODE_DIM), lambda i: (i, 0)),
            pl.BlockSpec((NODE_DIM, HIDDEN_DIM), lambda i: (0, 0)),
            pl.BlockSpec((1, HIDDEN_DIM), lambda i: (0, 0)),
            pl.BlockSpec((1, HIDDEN_DIM), lambda i: (0, 0)),
        ],
        out_specs=pl.BlockSpec((1, 1, _ROWS_PER_TILE), lambda i: (i, 0, 0)),
        out_shape=jax.ShapeDtypeStruct((_N_TILES, 1, _ROWS_PER_TILE), jnp.float32),
    )(x, W1, b1r, w2r)
    return out.reshape(N_NODES)


_NC = 2
_NS = 16
_NW = _NC * _NS
_L = 16
_CHUNK = -(-N_NODES // (_NW * _L)) * _L
_N_PAD = _CHUNK * _NW
_VECS = _CHUNK // _L


def _seg_body(val_hbm, idx_hbm, out_hbm, val_v, idx_v, acc_v, red_v, all_v,
              shared):
    cid = lax.axis_index("c")
    sid = lax.axis_index("s")
    wid = sid * _NC + cid
    base = wid * _CHUNK

    pltpu.sync_copy(val_hbm.at[pl.ds(base, _CHUNK)], val_v)
    pltpu.sync_copy(idx_hbm.at[pl.ds(base, _CHUNK)], idx_v)

    zeros = jnp.zeros((_L,), jnp.float32)

    def _zero(i, c):
        acc_v[pl.ds(i * _L, _L)] = zeros
        return c

    lax.fori_loop(0, (_L * NUM_SEGMENTS) // _L, _zero, 0)

    lane_off = lax.iota(jnp.int32, _L) * NUM_SEGMENTS

    def _scatter(i, c):
        v = val_v[pl.ds(i * _L, _L)]
        ix = idx_v[pl.ds(i * _L, _L)] + lane_off
        plsc.addupdate_scatter(acc_v, [ix], v)
        return c

    lax.fori_loop(0, _VECS, _scatter, 0)

    def _reduce(cb, c):
        t = acc_v[pl.ds(cb * _L, _L)]
        for r in range(1, _L):
            t = t + acc_v[pl.ds(r * NUM_SEGMENTS + cb * _L, _L)]
        red_v[pl.ds(cb * _L, _L)] = t
        return c

    lax.fori_loop(0, NUM_SEGMENTS // _L, _reduce, 0)

    pltpu.sync_copy(red_v, shared.at[pl.ds(sid * NUM_SEGMENTS, NUM_SEGMENTS)])
    plsc.subcore_barrier()

    @pl.when(sid == 0)
    def _combine():
        pltpu.sync_copy(shared, all_v)

        def _comb(cb, c):
            t = all_v[pl.ds(cb * _L, _L)]
            for r in range(1, _NS):
                t = t + all_v[pl.ds(r * NUM_SEGMENTS + cb * _L, _L)]
            red_v[pl.ds(cb * _L, _L)] = t
            return c

        lax.fori_loop(0, NUM_SEGMENTS // _L, _comb, 0)
        pltpu.sync_copy(red_v, out_hbm.at[cid])


@functools.lru_cache(maxsize=1)
def _build_segment_sum_sc():
    return pl.kernel(
        _seg_body,
        out_type=jax.ShapeDtypeStruct((_NC, NUM_SEGMENTS), jnp.float32),
        mesh=plsc.VectorSubcoreMesh(core_axis_name="c", subcore_axis_name="s"),
        compiler_params=pltpu.CompilerParams(needs_layout_passes=False),
        scratch_types=[
            pltpu.VMEM((_CHUNK,), jnp.float32),
            pltpu.VMEM((_CHUNK,), jnp.int32),
            pltpu.VMEM((_L * NUM_SEGMENTS,), jnp.float32),
            pltpu.VMEM((NUM_SEGMENTS,), jnp.float32),
            pltpu.VMEM((_NS * NUM_SEGMENTS,), jnp.float32),
            pltpu.VMEM_SHARED((_NS * NUM_SEGMENTS,), jnp.float32),
        ],
    )


def kernel(x, batch, W1, b1, W2):
    scal = _mlp_scalars(x, W1, b1, W2)
    scal = jnp.pad(scal, (0, _N_PAD - N_NODES))
    idx = jnp.pad(batch.astype(jnp.int32), (0, _N_PAD - N_NODES))
    partials = _build_segment_sum_sc()(scal, idx)
    return (partials[0] + partials[1]).reshape(NUM_SEGMENTS, 1)


# speedup vs baseline: 1.5285x; 1.0052x over previous
import functools

import jax
import jax.numpy as jnp
from jax import lax
from jax.experimental import pallas as pl
from jax.experimental.pallas import tpu as pltpu
from jax.experimental.pallas import tpu_sc as plsc

N_NODES = 100000
NODE_DIM = 128
HIDDEN_DIM = 128
NUM_SEGMENTS = 512

_LOG2 = 0.6931471805599453


_ROWS_PER_TILE = 2000
_N_TILES = N_NODES // _ROWS_PER_TILE


def _mlp_body(x_ref, w1_ref, b1_ref, w2_ref, o_ref):
    h = jnp.dot(x_ref[...], w1_ref[...], preferred_element_type=jnp.float32)
    h = h + b1_ref[...]
    a = jnp.maximum(h, 0.0) + jnp.log1p(jnp.exp(-jnp.abs(h))) - _LOG2
    o_ref[...] = jnp.dot(a, w2_ref[...], preferred_element_type=jnp.float32)[None]


def _mlp_scalars(x, W1, b1, W2):
    b1r = b1.reshape(1, HIDDEN_DIM)
    out = pl.pallas_call(
        _mlp_body,
        grid=(_N_TILES,),
        in_specs=[
            pl.BlockSpec((_ROWS_PER_TILE, NODE_DIM), lambda i: (i, 0)),
            pl.BlockSpec((NODE_DIM, HIDDEN_DIM), lambda i: (0, 0)),
            pl.BlockSpec((1, HIDDEN_DIM), lambda i: (0, 0)),
            pl.BlockSpec((HIDDEN_DIM, 1), lambda i: (0, 0)),
        ],
        out_specs=pl.BlockSpec((1, _ROWS_PER_TILE, 1), lambda i: (i, 0, 0)),
        out_shape=jax.ShapeDtypeStruct((_N_TILES, _ROWS_PER_TILE, 1), jnp.float32),
    )(x, W1, b1r, W2)
    return out.reshape(N_NODES)


_NC = 2
_NS = 16
_NW = _NC * _NS
_L = 16
_CHUNK = -(-N_NODES // (_NW * _L)) * _L
_N_PAD = _CHUNK * _NW
_VECS = _CHUNK // _L


def _seg_body(val_hbm, idx_hbm, out_hbm, val_v, idx_v, acc_v, red_v, all_v,
              shared):
    cid = lax.axis_index("c")
    sid = lax.axis_index("s")
    wid = sid * _NC + cid
    base = wid * _CHUNK

    pltpu.sync_copy(val_hbm.at[pl.ds(base, _CHUNK)], val_v)
    pltpu.sync_copy(idx_hbm.at[pl.ds(base, _CHUNK)], idx_v)

    zeros = jnp.zeros((_L,), jnp.float32)

    def _zero(i, c):
        acc_v[pl.ds(i * _L, _L)] = zeros
        return c

    lax.fori_loop(0, (_L * NUM_SEGMENTS) // _L, _zero, 0)

    lane_off = lax.iota(jnp.int32, _L) * NUM_SEGMENTS

    def _scatter(i, c):
        v = val_v[pl.ds(i * _L, _L)]
        ix = idx_v[pl.ds(i * _L, _L)] + lane_off
        plsc.addupdate_scatter(acc_v, [ix], v)
        return c

    lax.fori_loop(0, _VECS, _scatter, 0)

    def _reduce(cb, c):
        t = acc_v[pl.ds(cb * _L, _L)]
        for r in range(1, _L):
            t = t + acc_v[pl.ds(r * NUM_SEGMENTS + cb * _L, _L)]
        red_v[pl.ds(cb * _L, _L)] = t
        return c

    lax.fori_loop(0, NUM_SEGMENTS // _L, _reduce, 0)

    pltpu.sync_copy(red_v, shared.at[pl.ds(sid * NUM_SEGMENTS, NUM_SEGMENTS)])
    plsc.subcore_barrier()

    @pl.when(sid == 0)
    def _combine():
        pltpu.sync_copy(shared, all_v)

        def _comb(cb, c):
            t = all_v[pl.ds(cb * _L, _L)]
            for r in range(1, _NS):
                t = t + all_v[pl.ds(r * NUM_SEGMENTS + cb * _L, _L)]
            red_v[pl.ds(cb * _L, _L)] = t
            return c

        lax.fori_loop(0, NUM_SEGMENTS // _L, _comb, 0)
        pltpu.sync_copy(red_v, out_hbm.at[cid])


@functools.lru_cache(maxsize=1)
def _build_segment_sum_sc():
    return pl.kernel(
        _seg_body,
        out_type=jax.ShapeDtypeStruct((_NC, NUM_SEGMENTS), jnp.float32),
        mesh=plsc.VectorSubcoreMesh(core_axis_name="c", subcore_axis_name="s"),
        compiler_params=pltpu.CompilerParams(needs_layout_passes=False),
        scratch_types=[
            pltpu.VMEM((_CHUNK,), jnp.float32),
            pltpu.VMEM((_CHUNK,), jnp.int32),
            pltpu.VMEM((_L * NUM_SEGMENTS,), jnp.float32),
            pltpu.VMEM((NUM_SEGMENTS,), jnp.float32),
            pltpu.VMEM((_NS * NUM_SEGMENTS,), jnp.float32),
            pltpu.VMEM_SHARED((_NS * NUM_SEGMENTS,), jnp.float32),
        ],
    )


def kernel(x, batch, W1, b1, W2):
    scal = _mlp_scalars(x, W1, b1, W2)
    scal = jnp.pad(scal, (0, _N_PAD - N_NODES))
    idx = jnp.pad(batch.astype(jnp.int32), (0, _N_PAD - N_NODES))
    partials = _build_segment_sum_sc()(scal, idx)
    return (partials[0] + partials[1]).reshape(NUM_SEGMENTS, 1)


# speedup vs baseline: 2.0282x; 1.3270x over previous
import functools

import jax
import jax.numpy as jnp
from jax import lax
from jax.experimental import pallas as pl
from jax.experimental.pallas import tpu as pltpu
from jax.experimental.pallas import tpu_sc as plsc

N_NODES = 100000
NODE_DIM = 128
HIDDEN_DIM = 128
NUM_SEGMENTS = 512

_LOG2 = 0.6931471805599453


_ROWS_PER_TILE = 2000
_N_TILES = N_NODES // _ROWS_PER_TILE


def _mlp_body(x_ref, w1_ref, b1_ref, w2_ref, o_ref):
    h = jnp.dot(x_ref[...], w1_ref[...], preferred_element_type=jnp.float32)
    h = h + b1_ref[...]
    a = jnp.maximum(h, 0.0) + jnp.log1p(jnp.exp(-jnp.abs(h))) - _LOG2
    o = lax.dot_general(w2_ref[...], a, (((1,), (1,)), ((), ())),
                        preferred_element_type=jnp.float32)
    o_ref[...] = o[None]


def _mlp_scalars(x, W1, b1, W2):
    b1r = b1.reshape(1, HIDDEN_DIM)
    out = pl.pallas_call(
        _mlp_body,
        grid=(_N_TILES,),
        in_specs=[
            pl.BlockSpec((_ROWS_PER_TILE, NODE_DIM), lambda i: (i, 0)),
            pl.BlockSpec((NODE_DIM, HIDDEN_DIM), lambda i: (0, 0)),
            pl.BlockSpec((1, HIDDEN_DIM), lambda i: (0, 0)),
            pl.BlockSpec((1, HIDDEN_DIM), lambda i: (0, 0)),
        ],
        out_specs=pl.BlockSpec((1, 1, _ROWS_PER_TILE), lambda i: (i, 0, 0)),
        out_shape=jax.ShapeDtypeStruct((_N_TILES, 1, _ROWS_PER_TILE), jnp.float32),
    )(x, W1, b1r, W2.reshape(1, HIDDEN_DIM))
    return out.reshape(N_NODES)


_NC = 2
_NS = 16
_NW = _NC * _NS
_L = 16
_CHUNK = -(-N_NODES // (_NW * _L)) * _L
_N_PAD = _CHUNK * _NW
_VECS = _CHUNK // _L


def _seg_body(val_hbm, idx_hbm, out_hbm, val_v, idx_v, acc_v, red_v, all_v,
              shared):
    cid = lax.axis_index("c")
    sid = lax.axis_index("s")
    wid = sid * _NC + cid
    base = wid * _CHUNK

    pltpu.sync_copy(val_hbm.at[pl.ds(base, _CHUNK)], val_v)
    pltpu.sync_copy(idx_hbm.at[pl.ds(base, _CHUNK)], idx_v)

    zeros = jnp.zeros((_L,), jnp.float32)

    def _zero(i, c):
        acc_v[pl.ds(i * _L, _L)] = zeros
        return c

    lax.fori_loop(0, (_L * NUM_SEGMENTS) // _L, _zero, 0)

    lane_off = lax.iota(jnp.int32, _L) * NUM_SEGMENTS

    def _scatter(i, c):
        v = val_v[pl.ds(i * _L, _L)]
        ix = idx_v[pl.ds(i * _L, _L)] + lane_off
        plsc.addupdate_scatter(acc_v, [ix], v)
        return c

    lax.fori_loop(0, _VECS, _scatter, 0)

    def _reduce(cb, c):
        t = acc_v[pl.ds(cb * _L, _L)]
        for r in range(1, _L):
            t = t + acc_v[pl.ds(r * NUM_SEGMENTS + cb * _L, _L)]
        red_v[pl.ds(cb * _L, _L)] = t
        return c

    lax.fori_loop(0, NUM_SEGMENTS // _L, _reduce, 0)

    pltpu.sync_copy(red_v, shared.at[pl.ds(sid * NUM_SEGMENTS, NUM_SEGMENTS)])
    plsc.subcore_barrier()

    @pl.when(sid == 0)
    def _combine():
        pltpu.sync_copy(shared, all_v)

        def _comb(cb, c):
            t = all_v[pl.ds(cb * _L, _L)]
            for r in range(1, _NS):
                t = t + all_v[pl.ds(r * NUM_SEGMENTS + cb * _L, _L)]
            red_v[pl.ds(cb * _L, _L)] = t
            return c

        lax.fori_loop(0, NUM_SEGMENTS // _L, _comb, 0)
        pltpu.sync_copy(red_v, out_hbm.at[cid])


@functools.lru_cache(maxsize=1)
def _build_segment_sum_sc():
    return pl.kernel(
        _seg_body,
        out_type=jax.ShapeDtypeStruct((_NC, NUM_SEGMENTS), jnp.float32),
        mesh=plsc.VectorSubcoreMesh(core_axis_name="c", subcore_axis_name="s"),
        compiler_params=pltpu.CompilerParams(needs_layout_passes=False),
        scratch_types=[
            pltpu.VMEM((_CHUNK,), jnp.float32),
            pltpu.VMEM((_CHUNK,), jnp.int32),
            pltpu.VMEM((_L * NUM_SEGMENTS,), jnp.float32),
            pltpu.VMEM((NUM_SEGMENTS,), jnp.float32),
            pltpu.VMEM((_NS * NUM_SEGMENTS,), jnp.float32),
            pltpu.VMEM_SHARED((_NS * NUM_SEGMENTS,), jnp.float32),
        ],
    )


def kernel(x, batch, W1, b1, W2):
    scal = _mlp_scalars(x, W1, b1, W2)
    scal = jnp.pad(scal, (0, _N_PAD - N_NODES))
    idx = jnp.pad(batch.astype(jnp.int32), (0, _N_PAD - N_NODES))
    partials = _build_segment_sum_sc()(scal, idx)
    return (partials[0] + partials[1]).reshape(NUM_SEGMENTS, 1)


# speedup vs baseline: 2.4105x; 1.1885x over previous
import functools

import jax
import jax.numpy as jnp
from jax import lax
from jax.experimental import pallas as pl
from jax.experimental.pallas import tpu as pltpu
from jax.experimental.pallas import tpu_sc as plsc

N_NODES = 100000
NODE_DIM = 128
HIDDEN_DIM = 128
NUM_SEGMENTS = 512

_LOG2 = 0.6931471805599453


_ROWS_PER_TILE = 5000
_N_TILES = N_NODES // _ROWS_PER_TILE


def _mlp_body(x_ref, w1_ref, b1_ref, w2_ref, o_ref):
    h = jnp.dot(x_ref[...], w1_ref[...], preferred_element_type=jnp.float32)
    h = h + b1_ref[...]
    a = jnp.maximum(h, 0.0) + jnp.log1p(jnp.exp(-jnp.abs(h))) - _LOG2
    o = lax.dot_general(w2_ref[...], a, (((1,), (1,)), ((), ())),
                        preferred_element_type=jnp.float32)
    o_ref[...] = o[None]


def _mlp_scalars(x, W1, b1, W2):
    b1r = b1.reshape(1, HIDDEN_DIM)
    out = pl.pallas_call(
        _mlp_body,
        grid=(_N_TILES,),
        in_specs=[
            pl.BlockSpec((_ROWS_PER_TILE, NODE_DIM), lambda i: (i, 0)),
            pl.BlockSpec((NODE_DIM, HIDDEN_DIM), lambda i: (0, 0)),
            pl.BlockSpec((1, HIDDEN_DIM), lambda i: (0, 0)),
            pl.BlockSpec((1, HIDDEN_DIM), lambda i: (0, 0)),
        ],
        out_specs=pl.BlockSpec((1, 1, _ROWS_PER_TILE), lambda i: (i, 0, 0)),
        out_shape=jax.ShapeDtypeStruct((_N_TILES, 1, _ROWS_PER_TILE), jnp.float32),
    )(x, W1, b1r, W2.reshape(1, HIDDEN_DIM))
    return out.reshape(N_NODES)


_NC = 2
_NS = 16
_NW = _NC * _NS
_L = 16
_CHUNK = -(-N_NODES // (_NW * _L)) * _L
_N_PAD = _CHUNK * _NW
_VECS = _CHUNK // _L


def _seg_body(val_hbm, idx_hbm, out_hbm, val_v, idx_v, acc_v, red_v, all_v,
              shared):
    cid = lax.axis_index("c")
    sid = lax.axis_index("s")
    wid = sid * _NC + cid
    base = wid * _CHUNK

    pltpu.sync_copy(val_hbm.at[pl.ds(base, _CHUNK)], val_v)
    pltpu.sync_copy(idx_hbm.at[pl.ds(base, _CHUNK)], idx_v)

    zeros = jnp.zeros((_L,), jnp.float32)

    def _zero(i, c):
        acc_v[pl.ds(i * _L, _L)] = zeros
        return c

    lax.fori_loop(0, (_L * NUM_SEGMENTS) // _L, _zero, 0)

    lane_off = lax.iota(jnp.int32, _L) * NUM_SEGMENTS

    def _scatter(i, c):
        v = val_v[pl.ds(i * _L, _L)]
        ix = idx_v[pl.ds(i * _L, _L)] + lane_off
        plsc.addupdate_scatter(acc_v, [ix], v)
        return c

    lax.fori_loop(0, _VECS, _scatter, 0)

    def _reduce(cb, c):
        t = acc_v[pl.ds(cb * _L, _L)]
        for r in range(1, _L):
            t = t + acc_v[pl.ds(r * NUM_SEGMENTS + cb * _L, _L)]
        red_v[pl.ds(cb * _L, _L)] = t
        return c

    lax.fori_loop(0, NUM_SEGMENTS // _L, _reduce, 0)

    pltpu.sync_copy(red_v, shared.at[pl.ds(sid * NUM_SEGMENTS, NUM_SEGMENTS)])
    plsc.subcore_barrier()

    @pl.when(sid == 0)
    def _combine():
        pltpu.sync_copy(shared, all_v)

        def _comb(cb, c):
            t = all_v[pl.ds(cb * _L, _L)]
            for r in range(1, _NS):
                t = t + all_v[pl.ds(r * NUM_SEGMENTS + cb * _L, _L)]
            red_v[pl.ds(cb * _L, _L)] = t
            return c

        lax.fori_loop(0, NUM_SEGMENTS // _L, _comb, 0)
        pltpu.sync_copy(red_v, out_hbm.at[cid])


@functools.lru_cache(maxsize=1)
def _build_segment_sum_sc():
    return pl.kernel(
        _seg_body,
        out_type=jax.ShapeDtypeStruct((_NC, NUM_SEGMENTS), jnp.float32),
        mesh=plsc.VectorSubcoreMesh(core_axis_name="c", subcore_axis_name="s"),
        compiler_params=pltpu.CompilerParams(needs_layout_passes=False),
        scratch_types=[
            pltpu.VMEM((_CHUNK,), jnp.float32),
            pltpu.VMEM((_CHUNK,), jnp.int32),
            pltpu.VMEM((_L * NUM_SEGMENTS,), jnp.float32),
            pltpu.VMEM((NUM_SEGMENTS,), jnp.float32),
            pltpu.VMEM((_NS * NUM_SEGMENTS,), jnp.float32),
            pltpu.VMEM_SHARED((_NS * NUM_SEGMENTS,), jnp.float32),
        ],
    )


def kernel(x, batch, W1, b1, W2):
    scal = _mlp_scalars(x, W1, b1, W2)
    scal = jnp.pad(scal, (0, _N_PAD - N_NODES))
    idx = jnp.pad(batch.astype(jnp.int32), (0, _N_PAD - N_NODES))
    partials = _build_segment_sum_sc()(scal, idx)
    return (partials[0] + partials[1]).reshape(NUM_SEGMENTS, 1)


# speedup vs baseline: 2.6273x; 1.0899x over previous
import functools

import jax
import jax.numpy as jnp
from jax import lax
from jax.experimental import pallas as pl
from jax.experimental.pallas import tpu as pltpu
from jax.experimental.pallas import tpu_sc as plsc

N_NODES = 100000
NODE_DIM = 128
HIDDEN_DIM = 128
NUM_SEGMENTS = 512

_LOG2 = 0.6931471805599453


_ROWS_PER_TILE = 20000
_N_TILES = N_NODES // _ROWS_PER_TILE


def _mlp_body(x_ref, w1_ref, b1_ref, w2_ref, o_ref):
    h = jnp.dot(x_ref[...], w1_ref[...], preferred_element_type=jnp.float32)
    h = h + b1_ref[...]
    a = jnp.maximum(h, 0.0) + jnp.log1p(jnp.exp(-jnp.abs(h))) - _LOG2
    o = lax.dot_general(w2_ref[...], a, (((1,), (1,)), ((), ())),
                        preferred_element_type=jnp.float32)
    o_ref[...] = o[None]


def _mlp_scalars(x, W1, b1, W2):
    b1r = b1.reshape(1, HIDDEN_DIM)
    out = pl.pallas_call(
        _mlp_body,
        grid=(_N_TILES,),
        in_specs=[
            pl.BlockSpec((_ROWS_PER_TILE, NODE_DIM), lambda i: (i, 0)),
            pl.BlockSpec((NODE_DIM, HIDDEN_DIM), lambda i: (0, 0)),
            pl.BlockSpec((1, HIDDEN_DIM), lambda i: (0, 0)),
            pl.BlockSpec((1, HIDDEN_DIM), lambda i: (0, 0)),
        ],
        out_specs=pl.BlockSpec((1, 1, _ROWS_PER_TILE), lambda i: (i, 0, 0)),
        out_shape=jax.ShapeDtypeStruct((_N_TILES, 1, _ROWS_PER_TILE), jnp.float32),
    )(x, W1, b1r, W2.reshape(1, HIDDEN_DIM))
    return out.reshape(N_NODES)


_NC = 2
_NS = 16
_NW = _NC * _NS
_L = 16
_CHUNK = -(-N_NODES // (_NW * _L)) * _L
_N_PAD = _CHUNK * _NW
_VECS = _CHUNK // _L


def _seg_body(val_hbm, idx_hbm, out_hbm, val_v, idx_v, acc_v, red_v, all_v,
              shared):
    cid = lax.axis_index("c")
    sid = lax.axis_index("s")
    wid = sid * _NC + cid
    base = wid * _CHUNK

    pltpu.sync_copy(val_hbm.at[pl.ds(base, _CHUNK)], val_v)
    pltpu.sync_copy(idx_hbm.at[pl.ds(base, _CHUNK)], idx_v)

    zeros = jnp.zeros((_L,), jnp.float32)

    def _zero(i, c):
        acc_v[pl.ds(i * _L, _L)] = zeros
        return c

    lax.fori_loop(0, (_L * NUM_SEGMENTS) // _L, _zero, 0)

    lane_off = lax.iota(jnp.int32, _L) * NUM_SEGMENTS

    def _scatter(i, c):
        v = val_v[pl.ds(i * _L, _L)]
        ix = idx_v[pl.ds(i * _L, _L)] + lane_off
        plsc.addupdate_scatter(acc_v, [ix], v)
        return c

    lax.fori_loop(0, _VECS, _scatter, 0)

    def _reduce(cb, c):
        t = acc_v[pl.ds(cb * _L, _L)]
        for r in range(1, _L):
            t = t + acc_v[pl.ds(r * NUM_SEGMENTS + cb * _L, _L)]
        red_v[pl.ds(cb * _L, _L)] = t
        return c

    lax.fori_loop(0, NUM_SEGMENTS // _L, _reduce, 0)

    pltpu.sync_copy(red_v, shared.at[pl.ds(sid * NUM_SEGMENTS, NUM_SEGMENTS)])
    plsc.subcore_barrier()

    @pl.when(sid == 0)
    def _combine():
        pltpu.sync_copy(shared, all_v)

        def _comb(cb, c):
            t = all_v[pl.ds(cb * _L, _L)]
            for r in range(1, _NS):
                t = t + all_v[pl.ds(r * NUM_SEGMENTS + cb * _L, _L)]
            red_v[pl.ds(cb * _L, _L)] = t
            return c

        lax.fori_loop(0, NUM_SEGMENTS // _L, _comb, 0)
        pltpu.sync_copy(red_v, out_hbm.at[cid])


@functools.lru_cache(maxsize=1)
def _build_segment_sum_sc():
    return pl.kernel(
        _seg_body,
        out_type=jax.ShapeDtypeStruct((_NC, NUM_SEGMENTS), jnp.float32),
        mesh=plsc.VectorSubcoreMesh(core_axis_name="c", subcore_axis_name="s"),
        compiler_params=pltpu.CompilerParams(needs_layout_passes=False),
        scratch_types=[
            pltpu.VMEM((_CHUNK,), jnp.float32),
            pltpu.VMEM((_CHUNK,), jnp.int32),
            pltpu.VMEM((_L * NUM_SEGMENTS,), jnp.float32),
            pltpu.VMEM((NUM_SEGMENTS,), jnp.float32),
            pltpu.VMEM((_NS * NUM_SEGMENTS,), jnp.float32),
            pltpu.VMEM_SHARED((_NS * NUM_SEGMENTS,), jnp.float32),
        ],
    )


def kernel(x, batch, W1, b1, W2):
    scal = _mlp_scalars(x, W1, b1, W2)
    scal = jnp.pad(scal, (0, _N_PAD - N_NODES))
    idx = jnp.pad(batch.astype(jnp.int32), (0, _N_PAD - N_NODES))
    partials = _build_segment_sum_sc()(scal, idx)
    return (partials[0] + partials[1]).reshape(NUM_SEGMENTS, 1)


# speedup vs baseline: 2.6310x; 1.0014x over previous
import functools

import jax
import jax.numpy as jnp
from jax import lax
from jax.experimental import pallas as pl
from jax.experimental.pallas import tpu as pltpu
from jax.experimental.pallas import tpu_sc as plsc

N_NODES = 100000
NODE_DIM = 128
HIDDEN_DIM = 128
NUM_SEGMENTS = 512

_LOG2 = 0.6931471805599453


_ROWS_PER_TILE = 10000
_N_TILES = N_NODES // _ROWS_PER_TILE


def _mlp_body(x_ref, w1_ref, b1_ref, w2_ref, o_ref):
    h = jnp.dot(x_ref[...], w1_ref[...], preferred_element_type=jnp.float32)
    h = h + b1_ref[...]
    a = jnp.maximum(h, 0.0) + jnp.log1p(jnp.exp(-jnp.abs(h))) - _LOG2
    o = lax.dot_general(w2_ref[...], a, (((1,), (1,)), ((), ())),
                        preferred_element_type=jnp.float32)
    o_ref[...] = o[None]


def _mlp_scalars(x, W1, b1, W2):
    b1r = b1.reshape(1, HIDDEN_DIM)
    out = pl.pallas_call(
        _mlp_body,
        grid=(_N_TILES,),
        in_specs=[
            pl.BlockSpec((_ROWS_PER_TILE, NODE_DIM), lambda i: (i, 0)),
            pl.BlockSpec((NODE_DIM, HIDDEN_DIM), lambda i: (0, 0)),
            pl.BlockSpec((1, HIDDEN_DIM), lambda i: (0, 0)),
            pl.BlockSpec((1, HIDDEN_DIM), lambda i: (0, 0)),
        ],
        out_specs=pl.BlockSpec((1, 1, _ROWS_PER_TILE), lambda i: (i, 0, 0)),
        out_shape=jax.ShapeDtypeStruct((_N_TILES, 1, _ROWS_PER_TILE), jnp.float32),
    )(x, W1, b1r, W2.reshape(1, HIDDEN_DIM))
    return out.reshape(N_NODES)


_NC = 2
_NS = 16
_NW = _NC * _NS
_L = 16
_CHUNK = -(-N_NODES // (_NW * _L)) * _L
_N_PAD = _CHUNK * _NW
_VECS = _CHUNK // _L


def _seg_body(val_hbm, idx_hbm, out_hbm, val_v, idx_v, acc_v, red_v, all_v,
              shared):
    cid = lax.axis_index("c")
    sid = lax.axis_index("s")
    wid = sid * _NC + cid
    base = wid * _CHUNK

    pltpu.sync_copy(val_hbm.at[pl.ds(base, _CHUNK)], val_v)
    pltpu.sync_copy(idx_hbm.at[pl.ds(base, _CHUNK)], idx_v)

    zeros = jnp.zeros((_L,), jnp.float32)

    def _zero(i, c):
        acc_v[pl.ds(i * _L, _L)] = zeros
        return c

    lax.fori_loop(0, (_L * NUM_SEGMENTS) // _L, _zero, 0)

    lane_off = lax.iota(jnp.int32, _L) * NUM_SEGMENTS

    def _scatter(i, c):
        v = val_v[pl.ds(i * _L, _L)]
        ix = idx_v[pl.ds(i * _L, _L)] + lane_off
        plsc.addupdate_scatter(acc_v, [ix], v)
        return c

    lax.fori_loop(0, _VECS, _scatter, 0)

    def _reduce(cb, c):
        t = acc_v[pl.ds(cb * _L, _L)]
        for r in range(1, _L):
            t = t + acc_v[pl.ds(r * NUM_SEGMENTS + cb * _L, _L)]
        red_v[pl.ds(cb * _L, _L)] = t
        return c

    lax.fori_loop(0, NUM_SEGMENTS // _L, _reduce, 0)

    pltpu.sync_copy(red_v, shared.at[pl.ds(sid * NUM_SEGMENTS, NUM_SEGMENTS)])
    plsc.subcore_barrier()

    @pl.when(sid == 0)
    def _combine():
        pltpu.sync_copy(shared, all_v)

        def _comb(cb, c):
            t = all_v[pl.ds(cb * _L, _L)]
            for r in range(1, _NS):
                t = t + all_v[pl.ds(r * NUM_SEGMENTS + cb * _L, _L)]
            red_v[pl.ds(cb * _L, _L)] = t
            return c

        lax.fori_loop(0, NUM_SEGMENTS // _L, _comb, 0)
        pltpu.sync_copy(red_v, out_hbm.at[cid])


@functools.lru_cache(maxsize=1)
def _build_segment_sum_sc():
    return pl.kernel(
        _seg_body,
        out_type=jax.ShapeDtypeStruct((_NC, NUM_SEGMENTS), jnp.float32),
        mesh=plsc.VectorSubcoreMesh(core_axis_name="c", subcore_axis_name="s"),
        compiler_params=pltpu.CompilerParams(needs_layout_passes=False),
        scratch_types=[
            pltpu.VMEM((_CHUNK,), jnp.float32),
            pltpu.VMEM((_CHUNK,), jnp.int32),
            pltpu.VMEM((_L * NUM_SEGMENTS,), jnp.float32),
            pltpu.VMEM((NUM_SEGMENTS,), jnp.float32),
            pltpu.VMEM((_NS * NUM_SEGMENTS,), jnp.float32),
            pltpu.VMEM_SHARED((_NS * NUM_SEGMENTS,), jnp.float32),
        ],
    )


def kernel(x, batch, W1, b1, W2):
    scal = _mlp_scalars(x, W1, b1, W2)
    scal = jnp.pad(scal, (0, _N_PAD - N_NODES))
    idx = jnp.pad(batch.astype(jnp.int32), (0, _N_PAD - N_NODES))
    partials = _build_segment_sum_sc()(scal, idx)
    return (partials[0] + partials[1]).reshape(NUM_SEGMENTS, 1)


# speedup vs baseline: 2.8706x; 1.0910x over previous
import functools

import jax
import jax.numpy as jnp
from jax import lax
from jax.experimental import pallas as pl
from jax.experimental.pallas import tpu as pltpu
from jax.experimental.pallas import tpu_sc as plsc

N_NODES = 100000
NODE_DIM = 128
HIDDEN_DIM = 128
NUM_SEGMENTS = 512

_LOG2 = 0.6931471805599453


_ROWS_PER_TILE = 10000
_N_TILES = N_NODES // _ROWS_PER_TILE


def _mlp_body(x_ref, w1_ref, b1_ref, w2_ref, o_ref):
    h = jnp.dot(x_ref[...], w1_ref[...], preferred_element_type=jnp.float32)
    h = h + b1_ref[...]
    t = h * 1.4426950408889634
    a = (jnp.maximum(t, 0.0) + jnp.log2(1.0 + jnp.exp2(-jnp.abs(t))) - 1.0) * _LOG2
    o = lax.dot_general(w2_ref[...], a, (((1,), (1,)), ((), ())),
                        preferred_element_type=jnp.float32)
    o_ref[...] = o[None]


def _mlp_scalars(x, W1, b1, W2):
    b1r = b1.reshape(1, HIDDEN_DIM)
    out = pl.pallas_call(
        _mlp_body,
        grid=(_N_TILES,),
        in_specs=[
            pl.BlockSpec((_ROWS_PER_TILE, NODE_DIM), lambda i: (i, 0)),
            pl.BlockSpec((NODE_DIM, HIDDEN_DIM), lambda i: (0, 0)),
            pl.BlockSpec((1, HIDDEN_DIM), lambda i: (0, 0)),
            pl.BlockSpec((1, HIDDEN_DIM), lambda i: (0, 0)),
        ],
        out_specs=pl.BlockSpec((1, 1, _ROWS_PER_TILE), lambda i: (i, 0, 0)),
        out_shape=jax.ShapeDtypeStruct((_N_TILES, 1, _ROWS_PER_TILE), jnp.float32),
    )(x, W1, b1r, W2.reshape(1, HIDDEN_DIM))
    return out.reshape(N_NODES)


_NC = 2
_NS = 16
_NW = _NC * _NS
_L = 16
_CHUNK = -(-N_NODES // (_NW * _L)) * _L
_N_PAD = _CHUNK * _NW
_VECS = _CHUNK // _L


def _seg_body(val_hbm, idx_hbm, out_hbm, val_v, idx_v, acc_v, red_v, all_v,
              shared):
    cid = lax.axis_index("c")
    sid = lax.axis_index("s")
    wid = sid * _NC + cid
    base = wid * _CHUNK

    pltpu.sync_copy(val_hbm.at[pl.ds(base, _CHUNK)], val_v)
    pltpu.sync_copy(idx_hbm.at[pl.ds(base, _CHUNK)], idx_v)

    zeros = jnp.zeros((_L,), jnp.float32)

    def _zero(i, c):
        acc_v[pl.ds(i * _L, _L)] = zeros
        return c

    lax.fori_loop(0, (_L * NUM_SEGMENTS) // _L, _zero, 0)

    lane_off = lax.iota(jnp.int32, _L) * NUM_SEGMENTS

    def _scatter(i, c):
        v = val_v[pl.ds(i * _L, _L)]
        ix = idx_v[pl.ds(i * _L, _L)] + lane_off
        plsc.addupdate_scatter(acc_v, [ix], v)
        return c

    lax.fori_loop(0, _VECS, _scatter, 0)

    def _reduce(cb, c):
        t = acc_v[pl.ds(cb * _L, _L)]
        for r in range(1, _L):
            t = t + acc_v[pl.ds(r * NUM_SEGMENTS + cb * _L, _L)]
        red_v[pl.ds(cb * _L, _L)] = t
        return c

    lax.fori_loop(0, NUM_SEGMENTS // _L, _reduce, 0)

    pltpu.sync_copy(red_v, shared.at[pl.ds(sid * NUM_SEGMENTS, NUM_SEGMENTS)])
    plsc.subcore_barrier()

    @pl.when(sid == 0)
    def _combine():
        pltpu.sync_copy(shared, all_v)

        def _comb(cb, c):
            t = all_v[pl.ds(cb * _L, _L)]
            for r in range(1, _NS):
                t = t + all_v[pl.ds(r * NUM_SEGMENTS + cb * _L, _L)]
            red_v[pl.ds(cb * _L, _L)] = t
            return c

        lax.fori_loop(0, NUM_SEGMENTS // _L, _comb, 0)
        pltpu.sync_copy(red_v, out_hbm.at[cid])


@functools.lru_cache(maxsize=1)
def _build_segment_sum_sc():
    return pl.kernel(
        _seg_body,
        out_type=jax.ShapeDtypeStruct((_NC, NUM_SEGMENTS), jnp.float32),
        mesh=plsc.VectorSubcoreMesh(core_axis_name="c", subcore_axis_name="s"),
        compiler_params=pltpu.CompilerParams(needs_layout_passes=False),
        scratch_types=[
            pltpu.VMEM((_CHUNK,), jnp.float32),
            pltpu.VMEM((_CHUNK,), jnp.int32),
            pltpu.VMEM((_L * NUM_SEGMENTS,), jnp.float32),
            pltpu.VMEM((NUM_SEGMENTS,), jnp.float32),
            pltpu.VMEM((_NS * NUM_SEGMENTS,), jnp.float32),
            pltpu.VMEM_SHARED((_NS * NUM_SEGMENTS,), jnp.float32),
        ],
    )


def kernel(x, batch, W1, b1, W2):
    scal = _mlp_scalars(x, W1, b1, W2)
    scal = jnp.pad(scal, (0, _N_PAD - N_NODES))
    idx = jnp.pad(batch.astype(jnp.int32), (0, _N_PAD - N_NODES))
    partials = _build_segment_sum_sc()(scal, idx)
    return (partials[0] + partials[1]).reshape(NUM_SEGMENTS, 1)


# speedup vs baseline: 2.9850x; 1.0399x over previous
import functools

import jax
import jax.numpy as jnp
from jax import lax
from jax.experimental import pallas as pl
from jax.experimental.pallas import tpu as pltpu
from jax.experimental.pallas import tpu_sc as plsc

N_NODES = 100000
NODE_DIM = 128
HIDDEN_DIM = 128
NUM_SEGMENTS = 512

_LOG2 = 0.6931471805599453


_ROWS_PER_TILE = 10000
_N_TILES = N_NODES // _ROWS_PER_TILE


def _mlp_body(x_ref, w1_ref, b1_ref, w2_ref, o_ref):
    h = jnp.dot(x_ref[...], w1_ref[...], preferred_element_type=jnp.float32)
    h = h + b1_ref[...]
    t = h * 1.4426950408889634
    a = (jnp.maximum(t, 0.0) + jnp.log2(1.0 + jnp.exp2(-jnp.abs(t))) - 1.0) * _LOG2
    o = lax.dot_general(w2_ref[...], a, (((1,), (1,)), ((), ())),
                        preferred_element_type=jnp.float32)
    o_ref[...] = o[None]


def _mlp_scalars(x, W1, b1, W2):
    b1r = b1.reshape(1, HIDDEN_DIM)
    out = pl.pallas_call(
        _mlp_body,
        grid=(_N_TILES,),
        in_specs=[
            pl.BlockSpec((_ROWS_PER_TILE, NODE_DIM), lambda i: (i, 0)),
            pl.BlockSpec((NODE_DIM, HIDDEN_DIM), lambda i: (0, 0)),
            pl.BlockSpec((1, HIDDEN_DIM), lambda i: (0, 0)),
            pl.BlockSpec((1, HIDDEN_DIM), lambda i: (0, 0)),
        ],
        out_specs=pl.BlockSpec((1, 1, _ROWS_PER_TILE), lambda i: (i, 0, 0)),
        out_shape=jax.ShapeDtypeStruct((_N_TILES, 1, _ROWS_PER_TILE), jnp.float32),
    )(x, W1, b1r, W2.reshape(1, HIDDEN_DIM))
    return out.reshape(N_NODES)


_NC = 2
_NS = 16
_NW = _NC * _NS
_L = 16
_CHUNK = -(-N_NODES // (_NW * _L)) * _L
_N_PAD = _CHUNK * _NW
_VECS = _CHUNK // _L


def _seg_body(val_hbm, idx_hbm, out_hbm, val_v, idx_v, acc_v, red_v, all_v,
              shared):
    cid = lax.axis_index("c")
    sid = lax.axis_index("s")
    wid = sid * _NC + cid
    base = wid * _CHUNK

    pltpu.sync_copy(val_hbm.at[pl.ds(base, _CHUNK)], val_v)
    pltpu.sync_copy(idx_hbm.at[pl.ds(base, _CHUNK)], idx_v)

    zeros = jnp.zeros((_L,), jnp.float32)

    def _zero(i, c):
        for k in range(16):
            acc_v[pl.ds(i * (16 * _L) + k * _L, _L)] = zeros
        return c

    lax.fori_loop(0, (_L * NUM_SEGMENTS) // (16 * _L), _zero, 0)

    lane_off = lax.iota(jnp.int32, _L) * NUM_SEGMENTS

    def _scatter(i, c):
        for k in range(4):
            off = i * (4 * _L) + k * _L
            v = val_v[pl.ds(off, _L)]
            ix = idx_v[pl.ds(off, _L)] + lane_off
            plsc.addupdate_scatter(acc_v, [ix], v)
        return c

    lax.fori_loop(0, _VECS // 4, _scatter, 0)

    def _tree_sum(vs):
        while len(vs) > 1:
            vs = [vs[j] + vs[j + 1] for j in range(0, len(vs), 2)]
        return vs[0]

    def _reduce(cb, c):
        col = cb * _L
        red_v[pl.ds(col, _L)] = _tree_sum(
            [acc_v[pl.ds(r * NUM_SEGMENTS + col, _L)] for r in range(_L)])
        return c

    lax.fori_loop(0, NUM_SEGMENTS // _L, _reduce, 0)

    pltpu.sync_copy(red_v, shared.at[pl.ds(sid * NUM_SEGMENTS, NUM_SEGMENTS)])
    plsc.subcore_barrier()

    @pl.when(sid == 0)
    def _combine():
        pltpu.sync_copy(shared, all_v)

        def _comb(cb, c):
            col = cb * _L
            red_v[pl.ds(col, _L)] = _tree_sum(
                [all_v[pl.ds(r * NUM_SEGMENTS + col, _L)] for r in range(_NS)])
            return c

        lax.fori_loop(0, NUM_SEGMENTS // _L, _comb, 0)
        pltpu.sync_copy(red_v, out_hbm.at[cid])


@functools.lru_cache(maxsize=1)
def _build_segment_sum_sc():
    return pl.kernel(
        _seg_body,
        out_type=jax.ShapeDtypeStruct((_NC, NUM_SEGMENTS), jnp.float32),
        mesh=plsc.VectorSubcoreMesh(core_axis_name="c", subcore_axis_name="s"),
        compiler_params=pltpu.CompilerParams(needs_layout_passes=False),
        scratch_types=[
            pltpu.VMEM((_CHUNK,), jnp.float32),
            pltpu.VMEM((_CHUNK,), jnp.int32),
            pltpu.VMEM((_L * NUM_SEGMENTS,), jnp.float32),
            pltpu.VMEM((NUM_SEGMENTS,), jnp.float32),
            pltpu.VMEM((_NS * NUM_SEGMENTS,), jnp.float32),
            pltpu.VMEM_SHARED((_NS * NUM_SEGMENTS,), jnp.float32),
        ],
    )


def kernel(x, batch, W1, b1, W2):
    scal = _mlp_scalars(x, W1, b1, W2)
    scal = jnp.pad(scal, (0, _N_PAD - N_NODES))
    idx = jnp.pad(batch.astype(jnp.int32), (0, _N_PAD - N_NODES))
    partials = _build_segment_sum_sc()(scal, idx)
    return (partials[0] + partials[1]).reshape(NUM_SEGMENTS, 1)


# speedup vs baseline: 3.1836x; 1.0665x over previous
import functools

import jax
import jax.numpy as jnp
from jax import lax
from jax.experimental import pallas as pl
from jax.experimental.pallas import tpu as pltpu
from jax.experimental.pallas import tpu_sc as plsc

N_NODES = 100000
NODE_DIM = 128
HIDDEN_DIM = 128
NUM_SEGMENTS = 512

_LOG2 = 0.6931471805599453


_NC = 2
_NS = 16
_NW = _NC * _NS
_L = 16
_CHUNK = -(-N_NODES // (_NW * _L)) * _L
_N_PAD = _CHUNK * _NW
_VECS = _CHUNK // _L
_CHUNK_LAST = N_NODES - (_NW - 1) * _CHUNK
_VECS_LAST = _CHUNK_LAST // _L


_ROWS_PER_TILE = 12544
_N_TILES = _N_PAD // _ROWS_PER_TILE


def _mlp_body(x_ref, w1_ref, b1_ref, w2_ref, o_ref):
    i = pl.program_id(0)
    h = jnp.dot(x_ref[...], w1_ref[...], preferred_element_type=jnp.float32)
    h = h + b1_ref[...]
    t = h * 1.4426950408889634
    a = (jnp.maximum(t, 0.0) + jnp.log2(1.0 + jnp.exp2(-jnp.abs(t))) - 1.0) * _LOG2
    o = lax.dot_general(w2_ref[...], a, (((1,), (1,)), ((), ())),
                        preferred_element_type=jnp.float32)
    node = lax.broadcasted_iota(jnp.int32, (1, _ROWS_PER_TILE), 1)
    node = node + i * _ROWS_PER_TILE
    o_ref[...] = jnp.where(node < N_NODES, o, 0.0)


def _mlp_scalars(x, W1, b1, W2):
    b1r = b1.reshape(1, HIDDEN_DIM)
    out = pl.pallas_call(
        _mlp_body,
        grid=(_N_TILES,),
        in_specs=[
            pl.BlockSpec((_ROWS_PER_TILE, NODE_DIM), lambda i: (i, 0)),
            pl.BlockSpec((NODE_DIM, HIDDEN_DIM), lambda i: (0, 0)),
            pl.BlockSpec((1, HIDDEN_DIM), lambda i: (0, 0)),
            pl.BlockSpec((1, HIDDEN_DIM), lambda i: (0, 0)),
        ],
        out_specs=pl.BlockSpec((1, _ROWS_PER_TILE), lambda i: (0, i)),
        out_shape=jax.ShapeDtypeStruct((1, _N_PAD), jnp.float32),
    )(x, W1, b1r, W2.reshape(1, HIDDEN_DIM))
    return out.reshape(_N_PAD)


def _seg_body(val_hbm, idx_hbm, out_hbm, val_v, idx_v, acc_v, red_v, all_v,
              shared):
    cid = lax.axis_index("c")
    sid = lax.axis_index("s")
    wid = sid * _NC + cid
    base = wid * _CHUNK
    is_last = wid == _NW - 1

    pltpu.sync_copy(val_hbm.at[pl.ds(base, _CHUNK)], val_v)
    @pl.when(jnp.logical_not(is_last))
    def _full_idx():
        pltpu.sync_copy(idx_hbm.at[pl.ds(base, _CHUNK)], idx_v)

    @pl.when(is_last)
    def _tail_idx():
        pltpu.sync_copy(idx_hbm.at[pl.ds(base, _CHUNK_LAST)],
                        idx_v.at[pl.ds(0, _CHUNK_LAST)])

    zeros = jnp.zeros((_L,), jnp.float32)

    def _zero(i, c):
        for k in range(16):
            acc_v[pl.ds(i * (16 * _L) + k * _L, _L)] = zeros
        return c

    lax.fori_loop(0, (_L * NUM_SEGMENTS) // (16 * _L), _zero, 0)

    lane_off = lax.iota(jnp.int32, _L) * NUM_SEGMENTS

    def _scatter_one(vec):
        off = vec * _L
        v = val_v[pl.ds(off, _L)]
        ix = idx_v[pl.ds(off, _L)] + lane_off
        plsc.addupdate_scatter(acc_v, [ix], v)

    def _scatter(i, c):
        for k in range(4):
            _scatter_one(i * 4 + k)
        return c

    n4 = jnp.where(is_last, _VECS_LAST // 4, _VECS // 4)
    lax.fori_loop(0, n4, _scatter, 0)

    @pl.when(is_last)
    def _scatter_tail():
        for vec in range((_VECS_LAST // 4) * 4, _VECS_LAST):
            _scatter_one(vec)

    def _tree_sum(vs):
        while len(vs) > 1:
            vs = [vs[j] + vs[j + 1] for j in range(0, len(vs), 2)]
        return vs[0]

    def _reduce(cb, c):
        col = cb * _L
        red_v[pl.ds(col, _L)] = _tree_sum(
            [acc_v[pl.ds(r * NUM_SEGMENTS + col, _L)] for r in range(_L)])
        return c

    lax.fori_loop(0, NUM_SEGMENTS // _L, _reduce, 0)

    pltpu.sync_copy(red_v, shared.at[pl.ds(sid * NUM_SEGMENTS, NUM_SEGMENTS)])
    plsc.subcore_barrier()

    @pl.when(sid == 0)
    def _combine():
        pltpu.sync_copy(shared, all_v)

        def _comb(cb, c):
            col = cb * _L
            red_v[pl.ds(col, _L)] = _tree_sum(
                [all_v[pl.ds(r * NUM_SEGMENTS + col, _L)] for r in range(_NS)])
            return c

        lax.fori_loop(0, NUM_SEGMENTS // _L, _comb, 0)
        pltpu.sync_copy(red_v, out_hbm.at[cid])


@functools.lru_cache(maxsize=1)
def _build_segment_sum_sc():
    return pl.kernel(
        _seg_body,
        out_type=jax.ShapeDtypeStruct((_NC, NUM_SEGMENTS), jnp.float32),
        mesh=plsc.VectorSubcoreMesh(core_axis_name="c", subcore_axis_name="s"),
        compiler_params=pltpu.CompilerParams(needs_layout_passes=False),
        scratch_types=[
            pltpu.VMEM((_CHUNK,), jnp.float32),
            pltpu.VMEM((_CHUNK,), jnp.int32),
            pltpu.VMEM((_L * NUM_SEGMENTS,), jnp.float32),
            pltpu.VMEM((NUM_SEGMENTS,), jnp.float32),
            pltpu.VMEM((_NS * NUM_SEGMENTS,), jnp.float32),
            pltpu.VMEM_SHARED((_NS * NUM_SEGMENTS,), jnp.float32),
        ],
    )


def kernel(x, batch, W1, b1, W2):
    scal = _mlp_scalars(x, W1, b1, W2)
    idx = batch.astype(jnp.int32)
    partials = _build_segment_sum_sc()(scal, idx)
    return (partials[0] + partials[1]).reshape(NUM_SEGMENTS, 1)


# speedup vs baseline: 3.4004x; 1.0681x over previous
import functools

import jax
import jax.numpy as jnp
from jax import lax
from jax.experimental import pallas as pl
from jax.experimental.pallas import tpu as pltpu
from jax.experimental.pallas import tpu_sc as plsc

N_NODES = 100000
NODE_DIM = 128
HIDDEN_DIM = 128
NUM_SEGMENTS = 512

_LOG2 = 0.6931471805599453


_NC = 2
_NS = 16
_NW = _NC * _NS
_L = 16
_CHUNK = -(-N_NODES // (_NW * _L)) * _L
_N_PAD = _CHUNK * _NW
_VECS = _CHUNK // _L
_CHUNK_LAST = N_NODES - (_NW - 1) * _CHUNK
_VECS_LAST = _CHUNK_LAST // _L


_ROWS_PER_TILE = 12544
_N_TILES = _N_PAD // _ROWS_PER_TILE


def _mlp_body(x_ref, w1_ref, b1_ref, w2_ref, o_ref):
    i = pl.program_id(0)
    h = jnp.dot(x_ref[...], w1_ref[...], preferred_element_type=jnp.float32)
    h = h + b1_ref[...]
    t = h * 1.4426950408889634
    s = jnp.maximum(t, 0.0) + jnp.log2(1.0 + jnp.exp2(-jnp.abs(t)))
    w2s = w2_ref[...] * _LOG2
    o = lax.dot_general(w2s, s, (((1,), (1,)), ((), ())),
                        preferred_element_type=jnp.float32)
    o = o - jnp.sum(w2s)
    node = lax.broadcasted_iota(jnp.int32, (1, _ROWS_PER_TILE), 1)
    node = node + i * _ROWS_PER_TILE
    o_ref[...] = jnp.where(node < N_NODES, o, 0.0)


def _mlp_scalars(x, W1, b1, W2):
    b1r = b1.reshape(1, HIDDEN_DIM)
    out = pl.pallas_call(
        _mlp_body,
        grid=(_N_TILES,),
        in_specs=[
            pl.BlockSpec((_ROWS_PER_TILE, NODE_DIM), lambda i: (i, 0)),
            pl.BlockSpec((NODE_DIM, HIDDEN_DIM), lambda i: (0, 0)),
            pl.BlockSpec((1, HIDDEN_DIM), lambda i: (0, 0)),
            pl.BlockSpec((1, HIDDEN_DIM), lambda i: (0, 0)),
        ],
        out_specs=pl.BlockSpec((1, _ROWS_PER_TILE), lambda i: (0, i)),
        out_shape=jax.ShapeDtypeStruct((1, _N_PAD), jnp.float32),
    )(x, W1, b1r, W2.reshape(1, HIDDEN_DIM))
    return out.reshape(_N_PAD)


def _seg_body(val_hbm, idx_hbm, out_hbm, val_v, idx_v, acc_v, red_v, all_v,
              shared):
    cid = lax.axis_index("c")
    sid = lax.axis_index("s")
    wid = sid * _NC + cid
    base = wid * _CHUNK
    is_last = wid == _NW - 1

    pltpu.sync_copy(val_hbm.at[pl.ds(base, _CHUNK)], val_v)
    @pl.when(jnp.logical_not(is_last))
    def _full_idx():
        pltpu.sync_copy(idx_hbm.at[pl.ds(base, _CHUNK)], idx_v)

    @pl.when(is_last)
    def _tail_idx():
        pltpu.sync_copy(idx_hbm.at[pl.ds(base, _CHUNK_LAST)],
                        idx_v.at[pl.ds(0, _CHUNK_LAST)])

    zeros = jnp.zeros((_L,), jnp.float32)

    def _zero(i, c):
        for k in range(16):
            acc_v[pl.ds(i * (16 * _L) + k * _L, _L)] = zeros
        return c

    lax.fori_loop(0, (_L * NUM_SEGMENTS) // (16 * _L), _zero, 0)

    lane_off = lax.iota(jnp.int32, _L) * NUM_SEGMENTS

    def _scatter_one(vec):
        off = vec * _L
        v = val_v[pl.ds(off, _L)]
        ix = idx_v[pl.ds(off, _L)] + lane_off
        plsc.addupdate_scatter(acc_v, [ix], v)

    def _scatter(i, c):
        for k in range(4):
            _scatter_one(i * 4 + k)
        return c

    n4 = jnp.where(is_last, _VECS_LAST // 4, _VECS // 4)
    lax.fori_loop(0, n4, _scatter, 0)

    @pl.when(is_last)
    def _scatter_tail():
        for vec in range((_VECS_LAST // 4) * 4, _VECS_LAST):
            _scatter_one(vec)

    def _tree_sum(vs):
        while len(vs) > 1:
            vs = [vs[j] + vs[j + 1] for j in range(0, len(vs), 2)]
        return vs[0]

    def _reduce(cb, c):
        col = cb * _L
        red_v[pl.ds(col, _L)] = _tree_sum(
            [acc_v[pl.ds(r * NUM_SEGMENTS + col, _L)] for r in range(_L)])
        return c

    lax.fori_loop(0, NUM_SEGMENTS // _L, _reduce, 0)

    pltpu.sync_copy(red_v, shared.at[pl.ds(sid * NUM_SEGMENTS, NUM_SEGMENTS)])
    plsc.subcore_barrier()

    @pl.when(sid == 0)
    def _combine():
        pltpu.sync_copy(shared, all_v)

        def _comb(cb, c):
            col = cb * _L
            red_v[pl.ds(col, _L)] = _tree_sum(
                [all_v[pl.ds(r * NUM_SEGMENTS + col, _L)] for r in range(_NS)])
            return c

        lax.fori_loop(0, NUM_SEGMENTS // _L, _comb, 0)
        pltpu.sync_copy(red_v, out_hbm.at[cid])


@functools.lru_cache(maxsize=1)
def _build_segment_sum_sc():
    return pl.kernel(
        _seg_body,
        out_type=jax.ShapeDtypeStruct((_NC, NUM_SEGMENTS), jnp.float32),
        mesh=plsc.VectorSubcoreMesh(core_axis_name="c", subcore_axis_name="s"),
        compiler_params=pltpu.CompilerParams(needs_layout_passes=False),
        scratch_types=[
            pltpu.VMEM((_CHUNK,), jnp.float32),
            pltpu.VMEM((_CHUNK,), jnp.int32),
            pltpu.VMEM((_L * NUM_SEGMENTS,), jnp.float32),
            pltpu.VMEM((NUM_SEGMENTS,), jnp.float32),
            pltpu.VMEM((_NS * NUM_SEGMENTS,), jnp.float32),
            pltpu.VMEM_SHARED((_NS * NUM_SEGMENTS,), jnp.float32),
        ],
    )


def kernel(x, batch, W1, b1, W2):
    scal = _mlp_scalars(x, W1, b1, W2)
    idx = batch.astype(jnp.int32)
    partials = _build_segment_sum_sc()(scal, idx)
    return (partials[0] + partials[1]).reshape(NUM_SEGMENTS, 1)
